# baseline (speedup 1.0000x reference)
import functools

import jax
import jax.numpy as jnp
from jax import lax
from jax.experimental import pallas as pl
from jax.experimental.pallas import tpu as pltpu
from jax.experimental.pallas import tpu_sc as plsc

_ENC = 256.0


def _nll_body(x_ref, g_ref, o_ref):
    x = x_ref[0]
    g = g_ref[0]
    m = jnp.max(x, axis=0)
    s = jnp.sum(jnp.exp(x - m[None]), axis=0)
    cio = lax.broadcasted_iota(jnp.int32, x.shape, 0)
    gathered = jnp.sum(jnp.where(cio == g[None], x, 0.0), axis=0)
    nll = jnp.log(s) + m - gathered
    o_ref[0] = nll + 0.5 + g.astype(jnp.float32) * _ENC


def _nll_pass(net_output, gt):
    n, c, h, w = net_output.shape
    bh = 256
    grid = (n, h // bh)
    return pl.pallas_call(
        _nll_body,
        grid=grid,
        in_specs=[
            pl.BlockSpec((1, c, bh, w), lambda b, i: (b, 0, i, 0)),
            pl.BlockSpec((1, bh, w), lambda b, i: (b, i, 0)),
        ],
        out_specs=pl.BlockSpec((1, bh, w), lambda b, i: (b, i, 0)),
        out_shape=jax.ShapeDtypeStruct((n, h, w), jnp.float32),
    )(net_output, gt)


_L = 16
_NS = 16


def _sc_bin_body(num_classes, total, fused_hbm, part_hbm, out_hbm,
                 fv_v, sums_v, cnts_v, all_v, accs_v, accn_v, loss_v,
                 sem_0, sem_1):
    c = num_classes
    per_tile = total // _NS
    chunk = fv_v.shape[1]
    n_chunks = per_tile // chunk
    sid = lax.axis_index("s")
    base = sid * per_tile
    sems = (sem_0, sem_1)

    for i in range(c):
        sums_v[pl.ds(i * _L, _L)] = jnp.zeros((_L,), jnp.float32)
        cnts_v[pl.ds(i * _L, _L)] = jnp.zeros((_L,), jnp.float32)

    lane = lax.iota(jnp.int32, 16)
    ones = jnp.ones((_L,), jnp.float32)

    def copy(ch, b):
        src = pl.ds(base + ch * chunk, chunk)
        return pltpu.make_async_copy(fused_hbm.at[src], fv_v.at[b], sems[b])

    copy(0, 0).start()
    for ch in range(n_chunks):
        b = ch % 2
        if ch + 1 < n_chunks:
            copy(ch + 1, 1 - b).start()
        copy(ch, b).wait()

        @plsc.parallel_loop(0, chunk // _L, unroll=16)
        def _(i):
            v16 = fv_v[b, pl.ds(i * _L, _L)]
            g16 = (v16 * (1.0 / _ENC)).astype(jnp.int32)
            x16 = v16 - g16.astype(jnp.float32) * _ENC - 0.5
            flat = g16 * _L + lane
            plsc.addupdate_scatter(sums_v, [flat], x16)
            plsc.addupdate_scatter(cnts_v, [flat], ones)

    pltpu.sync_copy(sums_v, part_hbm.at[0, sid])
    pltpu.sync_copy(cnts_v, part_hbm.at[1, sid])
    plsc.subcore_barrier()

    @pl.when(sid == 0)
    def _():
        pltpu.sync_copy(part_hbm, all_v)
        for i in range(c):
            sv = jnp.zeros((_L,), jnp.float32)
            nv = jnp.zeros((_L,), jnp.float32)
            for t in range(_NS):
                sv = sv + all_v[0, t, pl.ds(i * _L, _L)]
                nv = nv + all_v[1, t, pl.ds(i * _L, _L)]
            accs_v[pl.ds(i * _L, _L)] = sv
            accn_v[pl.ds(i * _L, _L)] = nv
        cls = lax.iota(jnp.int32, 16)
        n_grp = (c + _L - 1) // _L
        num = jnp.float32(0.0)
        den = jnp.float32(0.0)
        for grp in range(n_grp):
            cid = cls + grp * _L
            valid = cid < c
            cidx = jnp.where(valid, cid, 0)
            s_vec = jnp.zeros((_L,), jnp.float32)
            n_vec = jnp.zeros((_L,), jnp.float32)
            for k in range(_L):
                s_vec = s_vec + plsc.load_gather(accs_v, [cidx * _L + k])
                n_vec = n_vec + plsc.load_gather(accn_v, [cidx * _L + k])
            present = jnp.logical_and(valid, n_vec > 0.0)
            ratio = jnp.where(present,
                              s_vec / jnp.maximum(n_vec, 1.0),
                              jnp.zeros((_L,), jnp.float32))
            num += jnp.sum(ratio)
            den += jnp.sum(jnp.where(present,
                                     jnp.ones((_L,), jnp.float32),
                                     jnp.zeros((_L,), jnp.float32)))
        loss_v[...] = (jnp.full((_L,), num, jnp.float32)
                       / jnp.full((_L,), den, jnp.float32))
        pltpu.sync_copy(loss_v, out_hbm)


def _sc_bin(fused_flat, num_classes):
    total = fused_flat.shape[0]
    mesh = plsc.VectorSubcoreMesh(
        core_axis_name="c", subcore_axis_name="s", num_cores=1)
    chunk = 32768
    c = num_classes
    kern = pl.kernel(
        functools.partial(_sc_bin_body, c, total),
        out_type=(
            jax.ShapeDtypeStruct((2, _NS, c * _L), jnp.float32),
            jax.ShapeDtypeStruct((_L,), jnp.float32),
        ),
        mesh=mesh,
        compiler_params=pltpu.CompilerParams(needs_layout_passes=False),
        scratch_types=[
            pltpu.VMEM((2, chunk), jnp.float32),
            pltpu.VMEM((c * _L,), jnp.float32),
            pltpu.VMEM((c * _L,), jnp.float32),
            pltpu.VMEM((2, _NS, c * _L), jnp.float32),
            pltpu.VMEM((c * _L,), jnp.float32),
            pltpu.VMEM((c * _L,), jnp.float32),
            pltpu.VMEM((_L,), jnp.float32),
            pltpu.SemaphoreType.DMA,
            pltpu.SemaphoreType.DMA,
        ],
    )
    _, loss16 = kern(fused_flat)
    return loss16


def kernel(net_output, gt):
    if net_output.ndim == gt.ndim:
        gt = gt[:, 0]
    num_classes = net_output.shape[1]
    fused = _nll_pass(net_output, gt)
    loss16 = _sc_bin(fused.reshape(-1), num_classes)
    return loss16[0]


# speedup vs baseline: 1.0014x; 1.0014x over previous
import functools

import jax
import jax.numpy as jnp
from jax import lax
from jax.experimental import pallas as pl
from jax.experimental.pallas import tpu as pltpu
from jax.experimental.pallas import tpu_sc as plsc

_ENC = 256.0


def _nll_body(x_ref, g_ref, o_ref):
    x = x_ref[0]
    g = g_ref[0]
    m = jnp.max(x, axis=0)
    s = jnp.sum(jnp.exp(x - m[None]), axis=0)
    cio = lax.broadcasted_iota(jnp.int32, x.shape, 0)
    gathered = jnp.sum(jnp.where(cio == g[None], x, 0.0), axis=0)
    nll = jnp.log(s) + m - gathered
    o_ref[0] = nll + 0.5 + g.astype(jnp.float32) * _ENC


def _nll_pass(net_output, gt):
    n, c, h, w = net_output.shape
    bh = 256
    grid = (n, h // bh)
    return pl.pallas_call(
        _nll_body,
        grid=grid,
        in_specs=[
            pl.BlockSpec((1, c, bh, w), lambda b, i: (b, 0, i, 0)),
            pl.BlockSpec((1, bh, w), lambda b, i: (b, i, 0)),
        ],
        out_specs=pl.BlockSpec((1, bh, w), lambda b, i: (b, i, 0)),
        out_shape=jax.ShapeDtypeStruct((n, h, w), jnp.float32),
    )(net_output, gt)


_L = 16
_NS = 16


def _sc_bin_body(num_classes, total, fused_hbm, part_hbm, out_hbm,
                 fv_v, sums_v, cnts_v, all_v, accs_v, accn_v, loss_v,
                 sem_0, sem_1):
    c = num_classes
    per_tile = total // _NS
    chunk = fv_v.shape[1]
    n_chunks = per_tile // chunk
    sid = lax.axis_index("s")
    base = sid * per_tile
    sems = (sem_0, sem_1)

    for i in range(c):
        sums_v[pl.ds(i * _L, _L)] = jnp.zeros((_L,), jnp.float32)
        cnts_v[pl.ds(i * _L, _L)] = jnp.zeros((_L,), jnp.float32)

    lane = lax.iota(jnp.int32, 16)
    ones = jnp.ones((_L,), jnp.float32)

    def copy(ch, b):
        src = pl.ds(base + ch * chunk, chunk)
        return pltpu.make_async_copy(fused_hbm.at[src], fv_v.at[b], sems[b])

    copy(0, 0).start()
    for ch in range(n_chunks):
        b = ch % 2
        if ch + 1 < n_chunks:
            copy(ch + 1, 1 - b).start()
        copy(ch, b).wait()

        @plsc.parallel_loop(0, chunk // _L, unroll=8)
        def _(i):
            v16 = fv_v[b, pl.ds(i * _L, _L)]
            g16 = (v16 * (1.0 / _ENC)).astype(jnp.int32)
            x16 = v16 - g16.astype(jnp.float32) * _ENC - 0.5
            flat = g16 * _L + lane
            plsc.addupdate_scatter(sums_v, [flat], x16)
            plsc.addupdate_scatter(cnts_v, [flat], ones)

    pltpu.sync_copy(sums_v, part_hbm.at[0, sid])
    pltpu.sync_copy(cnts_v, part_hbm.at[1, sid])
    plsc.subcore_barrier()

    @pl.when(sid == 0)
    def _():
        pltpu.sync_copy(part_hbm, all_v)
        for i in range(c):
            sv = jnp.zeros((_L,), jnp.float32)
            nv = jnp.zeros((_L,), jnp.float32)
            for t in range(_NS):
                sv = sv + all_v[0, t, pl.ds(i * _L, _L)]
                nv = nv + all_v[1, t, pl.ds(i * _L, _L)]
            accs_v[pl.ds(i * _L, _L)] = sv
            accn_v[pl.ds(i * _L, _L)] = nv
        cls = lax.iota(jnp.int32, 16)
        n_grp = (c + _L - 1) // _L
        num = jnp.float32(0.0)
        den = jnp.float32(0.0)
        for grp in range(n_grp):
            cid = cls + grp * _L
            valid = cid < c
            cidx = jnp.where(valid, cid, 0)
            s_vec = jnp.zeros((_L,), jnp.float32)
            n_vec = jnp.zeros((_L,), jnp.float32)
            for k in range(_L):
                s_vec = s_vec + plsc.load_gather(accs_v, [cidx * _L + k])
                n_vec = n_vec + plsc.load_gather(accn_v, [cidx * _L + k])
            present = jnp.logical_and(valid, n_vec > 0.0)
            ratio = jnp.where(present,
                              s_vec / jnp.maximum(n_vec, 1.0),
                              jnp.zeros((_L,), jnp.float32))
            num += jnp.sum(ratio)
            den += jnp.sum(jnp.where(present,
                                     jnp.ones((_L,), jnp.float32),
                                     jnp.zeros((_L,), jnp.float32)))
        loss_v[...] = (jnp.full((_L,), num, jnp.float32)
                       / jnp.full((_L,), den, jnp.float32))
        pltpu.sync_copy(loss_v, out_hbm)


def _sc_bin(fused_flat, num_classes):
    total = fused_flat.shape[0]
    mesh = plsc.VectorSubcoreMesh(
        core_axis_name="c", subcore_axis_name="s", num_cores=1)
    chunk = 32768
    c = num_classes
    kern = pl.kernel(
        functools.partial(_sc_bin_body, c, total),
        out_type=(
            jax.ShapeDtypeStruct((2, _NS, c * _L), jnp.float32),
            jax.ShapeDtypeStruct((_L,), jnp.float32),
        ),
        mesh=mesh,
        compiler_params=pltpu.CompilerParams(needs_layout_passes=False),
        scratch_types=[
            pltpu.VMEM((2, chunk), jnp.float32),
            pltpu.VMEM((c * _L,), jnp.float32),
            pltpu.VMEM((c * _L,), jnp.float32),
            pltpu.VMEM((2, _NS, c * _L), jnp.float32),
            pltpu.VMEM((c * _L,), jnp.float32),
            pltpu.VMEM((c * _L,), jnp.float32),
            pltpu.VMEM((_L,), jnp.float32),
            pltpu.SemaphoreType.DMA,
            pltpu.SemaphoreType.DMA,
        ],
    )
    _, loss16 = kern(fused_flat)
    return loss16


def kernel(net_output, gt):
    if net_output.ndim == gt.ndim:
        gt = gt[:, 0]
    num_classes = net_output.shape[1]
    fused = _nll_pass(net_output, gt)
    loss16 = _sc_bin(fused.reshape(-1), num_classes)
    return loss16[0]
